# baseline (speedup 1.0000x reference)
import functools

import jax
import jax.numpy as jnp
from jax import lax
from jax.experimental import pallas as pl
from jax.experimental.pallas import tpu as pltpu
from jax.experimental.pallas import tpu_sc as plsc

NU = 25000
NI = 25000
NN = NU + NI
NE = 800000
EMB = 64
HALF = EMB // 2
Q = 16

N2 = 51200
E2 = 819200
EPW = E2 // 16
B = 128
NB = EPW // B
CB = 8
NCH = NB // CB
NBUF = 4
RW = N2 // 16
CW = N2 // 2 // 16
TR = NN // 16


def _sc_aggregate(x8, src, dst2, z16, z1):
  mesh = plsc.VectorSubcoreMesh(core_axis_name="c", subcore_axis_name="s")

  @functools.partial(
      pl.kernel,
      mesh=mesh,
      out_type=[
          jax.ShapeDtypeStruct((N2, 8, Q), jnp.float32),
          jax.ShapeDtypeStruct((N2,), jnp.float32),
      ],
      scratch_types=[
          pltpu.VMEM((CB * B,), jnp.int32),
          pltpu.VMEM((CB * B,), jnp.int32),
          pltpu.VMEM((CB, B), jnp.int32),
          pltpu.VMEM((CB, B), jnp.int32),
          pltpu.VMEM((B, Q), jnp.float32),
          pltpu.VMEM((B, Q), jnp.float32),
          pltpu.VMEM((B, Q), jnp.float32),
          pltpu.VMEM((B, Q), jnp.float32),
          pltpu.VMEM((B,), jnp.float32),
          pltpu.VMEM_SHARED((N2, Q), jnp.float32),
          pltpu.VMEM_SHARED((N2, Q), jnp.float32),
          pltpu.VMEM_SHARED((N2,), jnp.float32),
          pltpu.SemaphoreType.DMA,
          pltpu.SemaphoreType.DMA,
          pltpu.SemaphoreType.DMA,
          pltpu.SemaphoreType.DMA,
          pltpu.SemaphoreType.DMA,
          pltpu.SemaphoreType.DMA,
          pltpu.SemaphoreType.DMA,
          pltpu.SemaphoreType.DMA,
          pltpu.SemaphoreType.DMA,
          pltpu.SemaphoreType.DMA,
          pltpu.SemaphoreType.DMA,
          pltpu.SemaphoreType.DMA,
          pltpu.SemaphoreType.DMA,
          pltpu.SemaphoreType.DMA,
      ],
      compiler_params=pltpu.CompilerParams(use_tc_tiling_on_sc=False),
  )
  def k(x8_hbm, src_hbm, dst2_hbm, z16_hbm, z1_hbm,
        agg_out, cnt_out, srcb0, srcb1, dstb0, dstb1, r0, r1, r2, r3,
        ones_v, tab_sh, acc_sh, cnt_sh,
        g0, g1, g2, g3, s0, s1, s2, s3, c0, c1, c2, c3, i0, i1):
    c = lax.axis_index("c")
    s = lax.axis_index("s")
    srcb = (srcb0, srcb1)
    dstb = (dstb0, dstb1)
    rows = (r0, r1, r2, r3)
    gsem = (g0, g1, g2, g3)
    ssem = (s0, s1, s2, s3)
    csem = (c0, c1, c2, c3)
    isem = (i0, i1)

    one16 = jnp.ones((16,), jnp.float32)
    for t in range(B // 16):
      ones_v[pl.ds(t * 16, 16)] = one16

    def ifire(kc, p):
      pltpu.async_copy(
          src_hbm.at[pl.ds(s * EPW + kc * CB * B, CB * B)], srcb[p], isem[p])
      pltpu.async_copy(
          dst2_hbm.at[pl.ds(s * NB + kc * CB, CB)], dstb[p], isem[p])

    def iwait(p):
      pltpu.make_async_copy(
          src_hbm.at[pl.ds(0, CB * B)], srcb[p], isem[p]).wait()
      pltpu.make_async_copy(
          dst2_hbm.at[pl.ds(0, CB)], dstb[p], isem[p]).wait()

    def run(qq, with_cnt):
      @pl.when(c == 0)
      def _():
        pltpu.sync_copy(x8_hbm.at[pl.ds(s * TR, TR), qq],
                        tab_sh.at[pl.ds(s * TR, TR)])

      @pl.when(c == 1)
      def _():
        pltpu.sync_copy(x8_hbm.at[pl.ds(s * TR, TR), 2 + qq],
                        tab_sh.at[pl.ds(s * TR, TR)])

      pltpu.sync_copy(z16_hbm.at[pl.ds(s * RW, RW)],
                      acc_sh.at[pl.ds(s * RW, RW)])
      if with_cnt:
        pltpu.sync_copy(z1_hbm.at[pl.ds(s * RW, RW)],
                        cnt_sh.at[pl.ds(s * RW, RW)])
      plsc.subcore_barrier()

      def gfire(src_c, la, b):
        pltpu.async_copy(
            tab_sh.at[src_c.at[pl.ds(la * B, B)]], rows[b], gsem[b])

      def gwait(b):
        pltpu.make_async_copy(
            tab_sh.at[srcb0.at[pl.ds(0, B)]], rows[b], gsem[b]).wait()

      def swait(b):
        pltpu.make_async_copy(
            rows[b], acc_sh.at[dstb0.at[0]], ssem[b]).wait()
        if with_cnt:
          pltpu.make_async_copy(
              ones_v, cnt_sh.at[dstb0.at[0]], csem[b]).wait()

      def chunk(kc, p):
        src_c, dst_c = srcb[p], dstb[p]
        src_n = srcb[1 - p]
        for la in range(CB):
          a = kc * CB + la
          b = la % NBUF
          b2 = (la + 2) % NBUF
          gwait(b)
          pltpu.async_copy(rows[b], acc_sh.at[dst_c.at[la]], ssem[b],
                           add=True)
          if with_cnt:
            pltpu.async_copy(ones_v, cnt_sh.at[dst_c.at[la]], csem[b],
                             add=True)

          @pl.when(a >= 2)
          def _():
            swait(b2)

          if la + 2 < CB:
            @pl.when(a + 2 < NB)
            def _():
              gfire(src_c, la + 2, b2)
          else:
            @pl.when(a + 2 < NB)
            def _():
              gfire(src_n, la + 2 - CB, b2)

          if la == 1:
            @pl.when((kc >= 1) & (kc + 1 < NCH))
            def _():
              ifire(kc + 1, 1 - p)
          if la == CB - 3:
            @pl.when(kc + 1 < NCH)
            def _():
              iwait(1 - p)

      ifire(0, 0)
      iwait(0)
      ifire(1, 1)
      gfire(srcb[0], 0, 0)
      gfire(srcb[0], 1, 1)

      def body(kp, carry):
        chunk(2 * kp, 0)
        chunk(2 * kp + 1, 1)
        return carry
      lax.fori_loop(0, NCH // 2, body, 0)

      swait((NB - 2) % NBUF)
      swait((NB - 1) % NBUF)
      plsc.subcore_barrier()

      @pl.when(c == 0)
      def _():
        pltpu.sync_copy(acc_sh.at[pl.ds(s * RW, RW)],
                        agg_out.at[pl.ds(s * RW, RW), qq])

      @pl.when(c == 1)
      def _():
        pltpu.sync_copy(acc_sh.at[pl.ds(s * RW, RW)],
                        agg_out.at[pl.ds(s * RW, RW), 2 + qq])

      if with_cnt:
        @pl.when(c == 0)
        def _():
          pltpu.sync_copy(cnt_sh.at[pl.ds(s * CW, CW)],
                          cnt_out.at[pl.ds(s * CW, CW)])

        @pl.when(c == 1)
        def _():
          pltpu.sync_copy(cnt_sh.at[pl.ds(N2 // 2 + s * CW, CW)],
                          cnt_out.at[pl.ds(N2 // 2 + s * CW, CW)])
      plsc.subcore_barrier()

    run(0, True)
    run(1, False)

  return k(x8, src, dst2, z16, z1)


def _tc_epilogue_body(x_ref, a_ref, cnt_ref,
                      wl_ref, bl_ref, wr_ref, out_ref):
  r = jnp.maximum(cnt_ref[...], 1.0)
  a = a_ref[:, :EMB] / r
  out_ref[...] = (
      jnp.dot(a, wl_ref[...], preferred_element_type=jnp.float32)
      + bl_ref[...]
      + jnp.dot(x_ref[...], wr_ref[...], preferred_element_type=jnp.float32)
  )


def _tc_epilogue(x_half, aggw, cnt2, W_l, b_l2, W_r, row0):
  BLK = 1000
  nblk = 25000 // BLK
  r0 = row0 // BLK
  return pl.pallas_call(
      _tc_epilogue_body,
      grid=(nblk,),
      in_specs=[
          pl.BlockSpec((BLK, EMB), lambda i: (i, 0)),
          pl.BlockSpec((BLK, 2 * EMB), lambda i, r0=r0: (r0 + i, 0)),
          pl.BlockSpec((BLK, 1), lambda i, r0=r0: (r0 + i, 0)),
          pl.BlockSpec((EMB, EMB), lambda i: (0, 0)),
          pl.BlockSpec((1, EMB), lambda i: (0, 0)),
          pl.BlockSpec((EMB, EMB), lambda i: (0, 0)),
      ],
      out_specs=pl.BlockSpec((BLK, EMB), lambda i: (i, 0)),
      out_shape=jax.ShapeDtypeStruct((25000, EMB), jnp.float32),
  )(x_half, aggw, cnt2, W_l, b_l2, W_r)


@jax.jit
def kernel(user_emb, item_emb, W_l, b_l, W_r, edge_index):
  xw = jnp.pad(jnp.concatenate([user_emb, item_emb], axis=0),
               ((0, 0), (0, EMB)))
  x8 = xw.reshape(NN, 8, Q)

  src = jnp.pad(edge_index[0], (0, E2 - NE))
  dst = jnp.pad(edge_index[1], (0, E2 - NE),
                constant_values=N2 - 1)
  dst2 = dst.reshape(E2 // B, B)

  z16 = jnp.zeros((N2, Q), jnp.float32)
  z1 = jnp.zeros((N2,), jnp.float32)

  agg8, cnt = _sc_aggregate(x8, src, dst2, z16, z1)
  aggw = agg8.reshape(N2, 8 * Q)
  cnt2 = cnt[:, None]
  b_l2 = b_l[None, :]

  out_u = _tc_epilogue(user_emb, aggw, cnt2, W_l, b_l2, W_r, 0)
  out_i = _tc_epilogue(item_emb, aggw, cnt2, W_l, b_l2, W_r, NU)
  return (out_u, out_i)


# speedup vs baseline: 1.5493x; 1.5493x over previous
import functools

import jax
import jax.numpy as jnp
from jax import lax
from jax.experimental import pallas as pl
from jax.experimental.pallas import tpu as pltpu
from jax.experimental.pallas import tpu_sc as plsc

NU = 25000
NI = 25000
NN = NU + NI
NE = 800000
EMB = 64
HALF = EMB // 2
Q = 16

N2 = 51200
E2 = 819200
EPW = E2 // 16
B = 128
NB = EPW // B
CB = 8
NCH = NB // CB
NBUF = 4
RW = N2 // 16
CW = N2 // 2 // 16
TR = NN // 16


def _sc_aggregate(xw, src, dst2, z16, z1):
  mesh = plsc.VectorSubcoreMesh(core_axis_name="c", subcore_axis_name="s")

  @functools.partial(
      pl.kernel,
      mesh=mesh,
      out_type=[
          jax.ShapeDtypeStruct((N2, 8 * Q), jnp.float32),
          jax.ShapeDtypeStruct((N2,), jnp.float32),
      ],
      scratch_types=[
          pltpu.VMEM((CB * B,), jnp.int32),
          pltpu.VMEM((CB * B,), jnp.int32),
          pltpu.VMEM((CB, B), jnp.int32),
          pltpu.VMEM((CB, B), jnp.int32),
          pltpu.VMEM((B, Q), jnp.float32),
          pltpu.VMEM((B, Q), jnp.float32),
          pltpu.VMEM((B, Q), jnp.float32),
          pltpu.VMEM((B, Q), jnp.float32),
          pltpu.VMEM((B,), jnp.float32),
          pltpu.VMEM_SHARED((N2, Q), jnp.float32),
          pltpu.VMEM_SHARED((N2, Q), jnp.float32),
          pltpu.VMEM_SHARED((N2,), jnp.float32),
          pltpu.SemaphoreType.DMA,
          pltpu.SemaphoreType.DMA,
          pltpu.SemaphoreType.DMA,
          pltpu.SemaphoreType.DMA,
          pltpu.SemaphoreType.DMA,
          pltpu.SemaphoreType.DMA,
          pltpu.SemaphoreType.DMA,
          pltpu.SemaphoreType.DMA,
          pltpu.SemaphoreType.DMA,
          pltpu.SemaphoreType.DMA,
          pltpu.SemaphoreType.DMA,
          pltpu.SemaphoreType.DMA,
          pltpu.SemaphoreType.DMA,
          pltpu.SemaphoreType.DMA,
      ],
      compiler_params=pltpu.CompilerParams(use_tc_tiling_on_sc=False),
  )
  def k(xw_hbm, src_hbm, dst2_hbm, z16_hbm, z1_hbm,
        agg_out, cnt_out, srcb0, srcb1, dstb0, dstb1, r0, r1, r2, r3,
        ones_v, tab_sh, acc_sh, cnt_sh,
        g0, g1, g2, g3, s0, s1, s2, s3, c0, c1, c2, c3, i0, i1):
    c = lax.axis_index("c")
    s = lax.axis_index("s")
    srcb = (srcb0, srcb1)
    dstb = (dstb0, dstb1)
    rows = (r0, r1, r2, r3)
    gsem = (g0, g1, g2, g3)
    ssem = (s0, s1, s2, s3)
    csem = (c0, c1, c2, c3)
    isem = (i0, i1)

    one16 = jnp.ones((16,), jnp.float32)
    for t in range(B // 16):
      ones_v[pl.ds(t * 16, 16)] = one16

    def ifire(kc, p):
      pltpu.async_copy(
          src_hbm.at[pl.ds(s * EPW + kc * CB * B, CB * B)], srcb[p], isem[p])
      pltpu.async_copy(
          dst2_hbm.at[pl.ds(s * NB + kc * CB, CB)], dstb[p], isem[p])

    def iwait(p):
      pltpu.make_async_copy(
          src_hbm.at[pl.ds(0, CB * B)], srcb[p], isem[p]).wait()
      pltpu.make_async_copy(
          dst2_hbm.at[pl.ds(0, CB)], dstb[p], isem[p]).wait()

    def run(qq, with_cnt):
      @pl.when(c == 0)
      def _():
        pltpu.sync_copy(xw_hbm.at[pl.ds(s * TR, TR), pl.ds(Q * qq, Q)],
                        tab_sh.at[pl.ds(s * TR, TR)])

      @pl.when(c == 1)
      def _():
        pltpu.sync_copy(xw_hbm.at[pl.ds(s * TR, TR), pl.ds(Q * (2 + qq), Q)],
                        tab_sh.at[pl.ds(s * TR, TR)])

      pltpu.sync_copy(z16_hbm.at[pl.ds(s * RW, RW)],
                      acc_sh.at[pl.ds(s * RW, RW)])
      if with_cnt:
        pltpu.sync_copy(z1_hbm.at[pl.ds(s * RW, RW)],
                        cnt_sh.at[pl.ds(s * RW, RW)])
      plsc.subcore_barrier()

      def gfire(src_c, la, b):
        pltpu.async_copy(
            tab_sh.at[src_c.at[pl.ds(la * B, B)]], rows[b], gsem[b])

      def gwait(b):
        pltpu.make_async_copy(
            tab_sh.at[srcb0.at[pl.ds(0, B)]], rows[b], gsem[b]).wait()

      def swait(b):
        pltpu.make_async_copy(
            rows[b], acc_sh.at[dstb0.at[0]], ssem[b]).wait()
        if with_cnt:
          pltpu.make_async_copy(
              ones_v, cnt_sh.at[dstb0.at[0]], csem[b]).wait()

      def chunk(kc, p):
        src_c, dst_c = srcb[p], dstb[p]
        src_n = srcb[1 - p]
        for la in range(CB):
          a = kc * CB + la
          b = la % NBUF
          b2 = (la + 2) % NBUF
          gwait(b)
          pltpu.async_copy(rows[b], acc_sh.at[dst_c.at[la]], ssem[b],
                           add=True)
          if with_cnt:
            pltpu.async_copy(ones_v, cnt_sh.at[dst_c.at[la]], csem[b],
                             add=True)

          @pl.when(a >= 2)
          def _():
            swait(b2)

          if la + 2 < CB:
            @pl.when(a + 2 < NB)
            def _():
              gfire(src_c, la + 2, b2)
          else:
            @pl.when(a + 2 < NB)
            def _():
              gfire(src_n, la + 2 - CB, b2)

          if la == 1:
            @pl.when((kc >= 1) & (kc + 1 < NCH))
            def _():
              ifire(kc + 1, 1 - p)
          if la == CB - 3:
            @pl.when(kc + 1 < NCH)
            def _():
              iwait(1 - p)

      ifire(0, 0)
      iwait(0)
      ifire(1, 1)
      gfire(srcb[0], 0, 0)
      gfire(srcb[0], 1, 1)

      def body(kp, carry):
        chunk(2 * kp, 0)
        chunk(2 * kp + 1, 1)
        return carry
      lax.fori_loop(0, NCH // 2, body, 0)

      swait((NB - 2) % NBUF)
      swait((NB - 1) % NBUF)
      plsc.subcore_barrier()

      @pl.when(c == 0)
      def _():
        pltpu.sync_copy(acc_sh.at[pl.ds(s * RW, RW)],
                        agg_out.at[pl.ds(s * RW, RW), pl.ds(Q * qq, Q)])

      @pl.when(c == 1)
      def _():
        pltpu.sync_copy(acc_sh.at[pl.ds(s * RW, RW)],
                        agg_out.at[pl.ds(s * RW, RW), pl.ds(Q * (2 + qq), Q)])

      if with_cnt:
        @pl.when(c == 0)
        def _():
          pltpu.sync_copy(cnt_sh.at[pl.ds(s * CW, CW)],
                          cnt_out.at[pl.ds(s * CW, CW)])

        @pl.when(c == 1)
        def _():
          pltpu.sync_copy(cnt_sh.at[pl.ds(N2 // 2 + s * CW, CW)],
                          cnt_out.at[pl.ds(N2 // 2 + s * CW, CW)])
      plsc.subcore_barrier()

    run(0, True)
    run(1, False)

  return k(xw, src, dst2, z16, z1)


def _tc_epilogue_body(x_ref, a_ref, cnt_ref,
                      wl_ref, bl_ref, wr_ref, out_ref):
  r = jnp.maximum(cnt_ref[...], 1.0)
  a = a_ref[:, :EMB] / r
  out_ref[...] = (
      jnp.dot(a, wl_ref[...], preferred_element_type=jnp.float32)
      + bl_ref[...]
      + jnp.dot(x_ref[...], wr_ref[...], preferred_element_type=jnp.float32)
  )


def _tc_epilogue(x_half, aggw, cnt2, W_l, b_l2, W_r, row0):
  BLK = 1000
  nblk = 25000 // BLK
  r0 = row0 // BLK
  return pl.pallas_call(
      _tc_epilogue_body,
      grid=(nblk,),
      in_specs=[
          pl.BlockSpec((BLK, EMB), lambda i: (i, 0)),
          pl.BlockSpec((BLK, 2 * EMB), lambda i, r0=r0: (r0 + i, 0)),
          pl.BlockSpec((BLK, 1), lambda i, r0=r0: (r0 + i, 0)),
          pl.BlockSpec((EMB, EMB), lambda i: (0, 0)),
          pl.BlockSpec((1, EMB), lambda i: (0, 0)),
          pl.BlockSpec((EMB, EMB), lambda i: (0, 0)),
      ],
      out_specs=pl.BlockSpec((BLK, EMB), lambda i: (i, 0)),
      out_shape=jax.ShapeDtypeStruct((25000, EMB), jnp.float32),
  )(x_half, aggw, cnt2, W_l, b_l2, W_r)


@jax.jit
def kernel(user_emb, item_emb, W_l, b_l, W_r, edge_index):
  xw = jnp.pad(jnp.concatenate([user_emb, item_emb], axis=0),
               ((0, 0), (0, EMB)))

  src = jnp.pad(edge_index[0], (0, E2 - NE))
  dst = jnp.pad(edge_index[1], (0, E2 - NE),
                constant_values=N2 - 1)
  dst2 = dst.reshape(E2 // B, B)

  z16 = jnp.zeros((N2, Q), jnp.float32)
  z1 = jnp.zeros((N2,), jnp.float32)

  aggw, cnt = _sc_aggregate(xw, src, dst2, z16, z1)
  cnt2 = cnt[:, None]
  b_l2 = b_l[None, :]

  out_u = _tc_epilogue(user_emb, aggw, cnt2, W_l, b_l2, W_r, 0)
  out_i = _tc_epilogue(item_emb, aggw, cnt2, W_l, b_l2, W_r, NU)
  return (out_u, out_i)


# speedup vs baseline: 1.5651x; 1.0102x over previous
import functools

import jax
import jax.numpy as jnp
from jax import lax
from jax.experimental import pallas as pl
from jax.experimental.pallas import tpu as pltpu
from jax.experimental.pallas import tpu_sc as plsc

NU = 25000
NI = 25000
NN = NU + NI
NE = 800000
EMB = 64
HALF = EMB // 2
Q = 16

N2 = 51200
E2 = 819200
EPW = E2 // 16
B = 128
NB = EPW // B
CB = 8
NCH = NB // CB
NBUF = 4
RW = N2 // 16
CW = N2 // 2 // 16
TR = NN // 16


def _sc_aggregate(xw, src, dst2, z16, z1):
  mesh = plsc.VectorSubcoreMesh(core_axis_name="c", subcore_axis_name="s")

  @functools.partial(
      pl.kernel,
      mesh=mesh,
      out_type=[
          jax.ShapeDtypeStruct((N2, 8 * Q), jnp.float32),
          jax.ShapeDtypeStruct((N2,), jnp.float32),
      ],
      scratch_types=[
          pltpu.VMEM((CB * B,), jnp.int32),
          pltpu.VMEM((CB * B,), jnp.int32),
          pltpu.VMEM((CB, B), jnp.int32),
          pltpu.VMEM((CB, B), jnp.int32),
          pltpu.VMEM((B, Q), jnp.float32),
          pltpu.VMEM((B, Q), jnp.float32),
          pltpu.VMEM((B, Q), jnp.float32),
          pltpu.VMEM((B, Q), jnp.float32),
          pltpu.VMEM((B,), jnp.float32),
          pltpu.VMEM_SHARED((N2, Q), jnp.float32),
          pltpu.VMEM_SHARED((N2, Q), jnp.float32),
          pltpu.VMEM_SHARED((N2,), jnp.float32),
          pltpu.SemaphoreType.DMA,
          pltpu.SemaphoreType.DMA,
          pltpu.SemaphoreType.DMA,
          pltpu.SemaphoreType.DMA,
          pltpu.SemaphoreType.DMA,
          pltpu.SemaphoreType.DMA,
          pltpu.SemaphoreType.DMA,
          pltpu.SemaphoreType.DMA,
          pltpu.SemaphoreType.DMA,
          pltpu.SemaphoreType.DMA,
          pltpu.SemaphoreType.DMA,
          pltpu.SemaphoreType.DMA,
          pltpu.SemaphoreType.DMA,
          pltpu.SemaphoreType.DMA,
      ],
      compiler_params=pltpu.CompilerParams(use_tc_tiling_on_sc=False),
  )
  def k(xw_hbm, src_hbm, dst2_hbm, z16_hbm, z1_hbm,
        agg_out, cnt_out, srcb0, srcb1, dstb0, dstb1, r0, r1, r2, r3,
        ones_v, tab_sh, acc_sh, cnt_sh,
        g0, g1, g2, g3, s0, s1, s2, s3, c0, c1, c2, c3, i0, i1):
    c = lax.axis_index("c")
    s = lax.axis_index("s")
    srcb = (srcb0, srcb1)
    dstb = (dstb0, dstb1)
    rows = (r0, r1, r2, r3)
    gsem = (g0, g1, g2, g3)
    ssem = (s0, s1, s2, s3)
    csem = (c0, c1, c2, c3)
    isem = (i0, i1)

    one16 = jnp.ones((16,), jnp.float32)
    for t in range(B // 16):
      ones_v[pl.ds(t * 16, 16)] = one16

    def ifire(kc, p):
      pltpu.async_copy(
          src_hbm.at[pl.ds(s * EPW + kc * CB * B, CB * B)], srcb[p], isem[p])
      pltpu.async_copy(
          dst2_hbm.at[pl.ds(s * NB + kc * CB, CB)], dstb[p], isem[p])

    def iwait(p):
      pltpu.make_async_copy(
          src_hbm.at[pl.ds(0, CB * B)], srcb[p], isem[p]).wait()
      pltpu.make_async_copy(
          dst2_hbm.at[pl.ds(0, CB)], dstb[p], isem[p]).wait()

    def run(qq, with_cnt):
      @pl.when(c == 0)
      def _():
        pltpu.sync_copy(xw_hbm.at[pl.ds(s * TR, TR), pl.ds(Q * qq, Q)],
                        tab_sh.at[pl.ds(s * TR, TR)])

      @pl.when(c == 1)
      def _():
        pltpu.sync_copy(xw_hbm.at[pl.ds(s * TR, TR), pl.ds(Q * (2 + qq), Q)],
                        tab_sh.at[pl.ds(s * TR, TR)])

      pltpu.sync_copy(z16_hbm.at[pl.ds(s * RW, RW)],
                      acc_sh.at[pl.ds(s * RW, RW)])
      if with_cnt:
        pltpu.sync_copy(z1_hbm.at[pl.ds(s * RW, RW)],
                        cnt_sh.at[pl.ds(s * RW, RW)])
      plsc.subcore_barrier()

      def gfire(src_c, la, b):
        pltpu.async_copy(
            tab_sh.at[src_c.at[pl.ds(la * B, B)]], rows[b], gsem[b])

      def gwait(b):
        pltpu.make_async_copy(
            tab_sh.at[srcb0.at[pl.ds(0, B)]], rows[b], gsem[b]).wait()

      def swait(b):
        pltpu.make_async_copy(
            rows[b], acc_sh.at[dstb0.at[0]], ssem[b]).wait()
        if with_cnt:
          pltpu.make_async_copy(
              ones_v, cnt_sh.at[dstb0.at[0]], csem[b]).wait()

      def chunk(kc, p):
        src_c, dst_c = srcb[p], dstb[p]
        src_n = srcb[1 - p]
        for la in range(CB):
          a = kc * CB + la
          b = la % NBUF
          b2 = (la + 2) % NBUF
          gwait(b)
          pltpu.async_copy(rows[b], acc_sh.at[dst_c.at[la]], ssem[b],
                           add=True)
          if with_cnt:
            pltpu.async_copy(ones_v, cnt_sh.at[dst_c.at[la]], csem[b],
                             add=True)

          @pl.when(a >= 2)
          def _():
            swait(b2)

          if la + 2 < CB:
            @pl.when(a + 2 < NB)
            def _():
              gfire(src_c, la + 2, b2)
          else:
            @pl.when(a + 2 < NB)
            def _():
              gfire(src_n, la + 2 - CB, b2)

          if la == 1:
            @pl.when((kc >= 1) & (kc + 1 < NCH))
            def _():
              ifire(kc + 1, 1 - p)
          if la == CB - 3:
            @pl.when(kc + 1 < NCH)
            def _():
              iwait(1 - p)

      ifire(0, 0)
      iwait(0)
      ifire(1, 1)
      gfire(srcb[0], 0, 0)
      gfire(srcb[0], 1, 1)

      def body(kp, carry):
        chunk(2 * kp, 0)
        chunk(2 * kp + 1, 1)
        return carry
      lax.fori_loop(0, NCH // 2, body, 0)

      swait((NB - 2) % NBUF)
      swait((NB - 1) % NBUF)
      plsc.subcore_barrier()

      @pl.when(c == 0)
      def _():
        pltpu.sync_copy(acc_sh.at[pl.ds(s * RW, RW)],
                        agg_out.at[pl.ds(s * RW, RW), pl.ds(Q * qq, Q)])

      @pl.when(c == 1)
      def _():
        pltpu.sync_copy(acc_sh.at[pl.ds(s * RW, RW)],
                        agg_out.at[pl.ds(s * RW, RW), pl.ds(Q * (2 + qq), Q)])

      if with_cnt:
        @pl.when(c == 0)
        def _():
          pltpu.sync_copy(cnt_sh.at[pl.ds(s * CW, CW)],
                          cnt_out.at[pl.ds(s * CW, CW)])

        @pl.when(c == 1)
        def _():
          pltpu.sync_copy(cnt_sh.at[pl.ds(N2 // 2 + s * CW, CW)],
                          cnt_out.at[pl.ds(N2 // 2 + s * CW, CW)])
      plsc.subcore_barrier()

    run(0, True)
    run(1, False)

  return k(xw, src, dst2, z16, z1)


def _tc_xwr_body(x_ref, wr_ref, bl_ref, out_ref):
  out_ref[...] = jnp.dot(
      x_ref[...], wr_ref[...], preferred_element_type=jnp.float32
  ) + bl_ref[...]


def _tc_xwr(x_half, W_r, b_l2):
  BLK = 1000
  return pl.pallas_call(
      _tc_xwr_body,
      grid=(25000 // BLK,),
      in_specs=[
          pl.BlockSpec((BLK, EMB), lambda i: (i, 0)),
          pl.BlockSpec((EMB, EMB), lambda i: (0, 0)),
          pl.BlockSpec((1, EMB), lambda i: (0, 0)),
      ],
      out_specs=pl.BlockSpec((BLK, EMB), lambda i: (i, 0)),
      out_shape=jax.ShapeDtypeStruct((25000, EMB), jnp.float32),
  )(x_half, W_r, b_l2)


def _tc_epilogue_body(xwr_ref, a_ref, cnt_ref, wl_ref, out_ref):
  r = jnp.maximum(cnt_ref[...], 1.0)
  a = a_ref[:, :EMB] / r
  out_ref[...] = jnp.dot(
      a, wl_ref[...], preferred_element_type=jnp.float32) + xwr_ref[...]


def _tc_epilogue(xwr, aggw, cnt2, W_l, row0):
  BLK = 1000
  nblk = 25000 // BLK
  r0 = row0 // BLK
  return pl.pallas_call(
      _tc_epilogue_body,
      grid=(nblk,),
      in_specs=[
          pl.BlockSpec((BLK, EMB), lambda i: (i, 0)),
          pl.BlockSpec((BLK, 2 * EMB), lambda i, r0=r0: (r0 + i, 0)),
          pl.BlockSpec((BLK, 1), lambda i, r0=r0: (r0 + i, 0)),
          pl.BlockSpec((EMB, EMB), lambda i: (0, 0)),
      ],
      out_specs=pl.BlockSpec((BLK, EMB), lambda i: (i, 0)),
      out_shape=jax.ShapeDtypeStruct((25000, EMB), jnp.float32),
  )(xwr, aggw, cnt2, W_l)


@jax.jit
def kernel(user_emb, item_emb, W_l, b_l, W_r, edge_index):
  xw = jnp.pad(jnp.concatenate([user_emb, item_emb], axis=0),
               ((0, 0), (0, EMB)))

  src = jnp.pad(edge_index[0], (0, E2 - NE))
  dst = jnp.pad(edge_index[1], (0, E2 - NE),
                constant_values=N2 - 1)
  dst2 = dst.reshape(E2 // B, B)

  z16 = jnp.zeros((N2, Q), jnp.float32)
  z1 = jnp.zeros((N2,), jnp.float32)

  aggw, cnt = _sc_aggregate(xw, src, dst2, z16, z1)
  cnt2 = cnt[:, None]
  b_l2 = b_l[None, :]

  xwr_u = _tc_xwr(user_emb, W_r, b_l2)
  xwr_i = _tc_xwr(item_emb, W_r, b_l2)
  out_u = _tc_epilogue(xwr_u, aggw, cnt2, W_l, 0)
  out_i = _tc_epilogue(xwr_i, aggw, cnt2, W_l, NU)
  return (out_u, out_i)
